# baseline (device time: 35600 ns/iter reference)
import jax
import jax.numpy as jnp
from jax import lax
from jax.experimental import pallas as pl
from jax.experimental.pallas import tpu as pltpu

N_DEV = 8

_OFFSETS = (
    (1, 1, 1),
    (1, 1, 0), (1, 0, 1), (0, 1, 1),
    (1, 0, 0), (0, 1, 0), (0, 0, 1),
)


def kernel(x, Win0, Wout0, Win1, Wout1, Win2, Wout2):
    M, D = x.shape
    CH = M // N_DEV
    bf16 = jnp.bfloat16

    def body(x_ref, win0, wout0, win1, wout1, win2, wout2, out_ref,
             xbuf, partial, rsbuf, rs_send, rs_recv, ag_send, ag_recv):
        me = lax.axis_index("i")

        mz = me // 4
        q = me % 4
        my_ = q // 2
        mx = (q % 2) ^ my_
        peers = []
        for (a, b, c) in _OFFSETS:
            tx, ty, tz = mx ^ a, my_ ^ b, mz ^ c
            peers.append(tz * 4 + 2 * ty + (tx ^ ty))

        barrier = pltpu.get_barrier_semaphore()
        for p in peers:
            pl.semaphore_signal(
                barrier, inc=1,
                device_id=(p,), device_id_type=pl.DeviceIdType.MESH,
            )
        pl.semaphore_wait(barrier, N_DEV - 1)

        def rs_copy(p):
            return pltpu.make_async_remote_copy(
                src_ref=partial.at[pl.ds(p * CH, CH), :],
                dst_ref=rsbuf.at[pl.ds(me * CH, CH), :],
                send_sem=rs_send.at[p],
                recv_sem=rs_recv.at[me],
                device_id=(p,),
                device_id_type=pl.DeviceIdType.MESH,
            )

        def rs_wait_from(s):
            return pltpu.make_async_remote_copy(
                src_ref=partial.at[pl.ds(0, CH), :],
                dst_ref=rsbuf.at[pl.ds(s * CH, CH), :],
                send_sem=rs_send.at[s],
                recv_sem=rs_recv.at[s],
                device_id=(s,),
                device_id_type=pl.DeviceIdType.MESH,
            )

        def ag_copy(p):
            return pltpu.make_async_remote_copy(
                src_ref=xbuf.at[pl.ds(me * CH, CH), :],
                dst_ref=xbuf.at[pl.ds(me * CH, CH), :],
                send_sem=ag_send.at[p],
                recv_sem=ag_recv.at[me],
                device_id=(p,),
                device_id_type=pl.DeviceIdType.MESH,
            )

        def ag_wait_from(s):
            return pltpu.make_async_remote_copy(
                src_ref=xbuf.at[pl.ds(0, CH), :],
                dst_ref=xbuf.at[pl.ds(s * CH, CH), :],
                send_sem=ag_send.at[s],
                recv_sem=ag_recv.at[s],
                device_id=(s,),
                device_id_type=pl.DeviceIdType.MESH,
            )

        wins = [win0, win1, win2]
        wouts = [wout0, wout1, wout2]

        for l in range(3):
            with jax.named_scope(f"compute#l={l}"):
                xv = x_ref[...].astype(bf16) if l == 0 else xbuf[...]
                h = jnp.dot(xv, wins[l][...].astype(bf16),
                            preferred_element_type=jnp.float32)
                h = jnp.maximum(h, 0.0).astype(bf16)
                pv = jnp.dot(h, wouts[l][...].astype(bf16),
                             preferred_element_type=jnp.float32)
                if l > 0:
                    for p in peers:
                        rs_copy(p).wait_send()
                partial[...] = pv.astype(bf16)

            with jax.named_scope(f"rs#l={l}"):
                for p in peers:
                    rs_copy(p).start()
                for s in peers:
                    rs_wait_from(s).wait_recv()

            with jax.named_scope(f"reduce#l={l}"):
                acc = partial[pl.ds(me * CH, CH), :].astype(jnp.float32)
                for s in peers:
                    acc = acc + rsbuf[pl.ds(s * CH, CH), :].astype(jnp.float32)
                if l > 0:
                    for p in peers:
                        ag_copy(p).wait_send()
                xbuf[pl.ds(me * CH, CH), :] = acc.astype(bf16)

            with jax.named_scope(f"ag#l={l}"):
                for p in peers:
                    ag_copy(p).start()
                for s in peers:
                    ag_wait_from(s).wait_recv()

        with jax.named_scope("out_drain"):
            out_ref[...] = xbuf[...].astype(jnp.float32)
            for p in peers:
                rs_copy(p).wait_send()
                ag_copy(p).wait_send()

    return pl.pallas_call(
        body,
        out_shape=jax.ShapeDtypeStruct((M, D), jnp.float32),
        in_specs=[pl.BlockSpec(memory_space=pltpu.VMEM)] * 7,
        out_specs=pl.BlockSpec(memory_space=pltpu.VMEM),
        scratch_shapes=[
            pltpu.VMEM((M, D), bf16),
            pltpu.VMEM((M, D), bf16),
            pltpu.VMEM((M, D), bf16),
            pltpu.SemaphoreType.DMA((N_DEV,)),
            pltpu.SemaphoreType.DMA((N_DEV,)),
            pltpu.SemaphoreType.DMA((N_DEV,)),
            pltpu.SemaphoreType.DMA((N_DEV,)),
        ],
        compiler_params=pltpu.CompilerParams(collective_id=0),
    )(x, Win0, Wout0, Win1, Wout1, Win2, Wout2)


# device time: 33468 ns/iter; 1.0637x vs baseline; 1.0637x over previous
import jax
import jax.numpy as jnp
from jax import lax
from jax.experimental import pallas as pl
from jax.experimental.pallas import tpu as pltpu

N_DEV = 8
N_STREAMS = 2

_OFFSETS = (
    (1, 1, 1),
    (1, 1, 0), (1, 0, 1), (0, 1, 1),
    (1, 0, 0), (0, 1, 0), (0, 0, 1),
)


def kernel(x, Win0, Wout0, Win1, Wout1, Win2, Wout2):
    M, D = x.shape
    F = Win0.shape[1]
    R = M // N_STREAMS
    CH = R // N_DEV
    bf16 = jnp.bfloat16

    def body(x_hbm, win0, wout0, win1, wout1, win2, wout2, out_hbm,
             xbuf, partial, rsbuf, xin, winv, woutv, outstage,
             rs_send, rs_recv, ag_send, ag_recv, load_sems):
        me = lax.axis_index("i")

        in_load = [
            pltpu.make_async_copy(x_hbm, xin, load_sems.at[0]),
            pltpu.make_async_copy(win0, winv.at[0], load_sems.at[1]),
            pltpu.make_async_copy(wout0, woutv.at[0], load_sems.at[2]),
            pltpu.make_async_copy(win1, winv.at[1], load_sems.at[3]),
            pltpu.make_async_copy(wout1, woutv.at[1], load_sems.at[4]),
            pltpu.make_async_copy(win2, winv.at[2], load_sems.at[5]),
            pltpu.make_async_copy(wout2, woutv.at[2], load_sems.at[6]),
        ]
        for c in in_load:
            c.start()

        mz = me // 4
        q = me % 4
        my_ = q // 2
        mx = (q % 2) ^ my_
        peers = []
        for (a, b, c) in _OFFSETS:
            tx, ty, tz = mx ^ a, my_ ^ b, mz ^ c
            peers.append(tz * 4 + 2 * ty + (tx ^ ty))

        barrier = pltpu.get_barrier_semaphore()
        for p in peers:
            pl.semaphore_signal(
                barrier, inc=1,
                device_id=(p,), device_id_type=pl.DeviceIdType.MESH,
            )
        pl.semaphore_wait(barrier, N_DEV - 1)

        def row0(t, j):
            return t * R + j * CH

        def rs_copy(t, p):
            return pltpu.make_async_remote_copy(
                src_ref=partial.at[pl.ds(row0(t, p), CH), :],
                dst_ref=rsbuf.at[pl.ds(row0(t, me), CH), :],
                send_sem=rs_send.at[t * N_DEV + p],
                recv_sem=rs_recv.at[t * N_DEV + me],
                device_id=(p,),
                device_id_type=pl.DeviceIdType.MESH,
            )

        def rs_wait_from(t, s):
            return pltpu.make_async_remote_copy(
                src_ref=partial.at[pl.ds(0, CH), :],
                dst_ref=rsbuf.at[pl.ds(row0(t, s), CH), :],
                send_sem=rs_send.at[t * N_DEV + s],
                recv_sem=rs_recv.at[t * N_DEV + s],
                device_id=(s,),
                device_id_type=pl.DeviceIdType.MESH,
            )

        def ag_copy(t, p):
            return pltpu.make_async_remote_copy(
                src_ref=xbuf.at[pl.ds(row0(t, me), CH), :],
                dst_ref=xbuf.at[pl.ds(row0(t, me), CH), :],
                send_sem=ag_send.at[t * N_DEV + p],
                recv_sem=ag_recv.at[t * N_DEV + me],
                device_id=(p,),
                device_id_type=pl.DeviceIdType.MESH,
            )

        def ag_wait_from(t, s):
            return pltpu.make_async_remote_copy(
                src_ref=xbuf.at[pl.ds(0, CH), :],
                dst_ref=xbuf.at[pl.ds(row0(t, s), CH), :],
                send_sem=ag_send.at[t * N_DEV + s],
                recv_sem=ag_recv.at[t * N_DEV + s],
                device_id=(s,),
                device_id_type=pl.DeviceIdType.MESH,
            )

        def compute_stream(t, l, from_input):
            rows = slice(t * R, (t + 1) * R)
            xv = (xin[rows, :].astype(bf16) if from_input
                  else xbuf[rows, :])
            h = jnp.dot(xv, winv[l].astype(bf16),
                        preferred_element_type=jnp.float32)
            h = jnp.maximum(h, 0.0).astype(bf16)
            pv = jnp.dot(h, woutv[l].astype(bf16),
                         preferred_element_type=jnp.float32)
            return pv.astype(bf16)

        def reduce_stream(t):
            acc = partial[pl.ds(row0(t, me), CH), :].astype(jnp.float32)
            for s in peers:
                acc = acc + rsbuf[pl.ds(row0(t, s), CH), :].astype(jnp.float32)
            return acc.astype(bf16)

        for l in range(3):
            with jax.named_scope(f"compute_rs#l={l}"):
                if l == 0:
                    in_load[0].wait()
                in_load[2 * l + 1].wait()
                in_load[2 * l + 2].wait()
                for t in range(N_STREAMS):
                    if l > 0:
                        for s in peers:
                            ag_wait_from(t, s).wait_recv()
                        for p in peers:
                            rs_copy(t, p).wait_send()
                    pv = compute_stream(t, l, from_input=(l == 0))
                    partial[t * R:(t + 1) * R, :] = pv
                    for p in peers:
                        rs_copy(t, p).start()

            with jax.named_scope(f"reduce_ag#l={l}"):
                for t in range(N_STREAMS):
                    for s in peers:
                        rs_wait_from(t, s).wait_recv()
                    red = reduce_stream(t)
                    if l > 0:
                        for p in peers:
                            ag_copy(t, p).wait_send()
                    xbuf[pl.ds(row0(t, me), CH), :] = red
                    for p in peers:
                        ag_copy(t, p).start()

        with jax.named_scope("out_drain"):
            for t in range(N_STREAMS):
                for s in peers:
                    ag_wait_from(t, s).wait_recv()
            outstage[...] = xbuf[...].astype(jnp.float32)
            out_copy = pltpu.make_async_copy(outstage, out_hbm,
                                             load_sems.at[7])
            out_copy.start()
            for t in range(N_STREAMS):
                for p in peers:
                    rs_copy(t, p).wait_send()
                    ag_copy(t, p).wait_send()
            out_copy.wait()

    return pl.pallas_call(
        body,
        out_shape=jax.ShapeDtypeStruct((M, D), jnp.float32),
        in_specs=[pl.BlockSpec(memory_space=pl.ANY)] * 7,
        out_specs=pl.BlockSpec(memory_space=pl.ANY),
        scratch_shapes=[
            pltpu.VMEM((M, D), bf16),
            pltpu.VMEM((M, D), bf16),
            pltpu.VMEM((M, D), bf16),
            pltpu.VMEM((M, D), jnp.float32),
            pltpu.VMEM((3, D, F), jnp.float32),
            pltpu.VMEM((3, F, D), jnp.float32),
            pltpu.VMEM((M, D), jnp.float32),
            pltpu.SemaphoreType.DMA((N_STREAMS * N_DEV,)),
            pltpu.SemaphoreType.DMA((N_STREAMS * N_DEV,)),
            pltpu.SemaphoreType.DMA((N_STREAMS * N_DEV,)),
            pltpu.SemaphoreType.DMA((N_STREAMS * N_DEV,)),
            pltpu.SemaphoreType.DMA((8,)),
        ],
        compiler_params=pltpu.CompilerParams(collective_id=0),
    )(x, Win0, Wout0, Win1, Wout1, Win2, Wout2)
